# Initial kernel scaffold; baseline (speedup 1.0000x reference)
import jax
import jax.numpy as jnp
from jax.experimental import pallas as pl


def kernel(x, pos, token_table, pos_table):
    raise NotImplementedError("write your pallas kernel here")


# speedup vs baseline: 5.5219x; 5.5219x over previous
import functools

import jax
import jax.numpy as jnp
from jax import lax
from jax.experimental import pallas as pl
from jax.experimental.pallas import tpu as pltpu
from jax.experimental.pallas import tpu_sc as plsc

V, D, P = 1000, 64, 512
B, L = 4096, 200
NC, NS = 2, 16
NW = NC * NS
N = B * L
TPW = N // NW
G = 128
NG = TPW // G

_mesh = plsc.VectorSubcoreMesh(core_axis_name="c", subcore_axis_name="s")


@functools.partial(
    pl.kernel,
    mesh=_mesh,
    compiler_params=pltpu.CompilerParams(use_tc_tiling_on_sc=False),
    out_type=jax.ShapeDtypeStruct((N, D), jnp.float32),
    scratch_types=[
        pltpu.VMEM((NG, G), jnp.int32),
        pltpu.VMEM((NG, G), jnp.int32),
        pltpu.VMEM((G, D), jnp.float32),
        pltpu.VMEM((G, D), jnp.float32),
        pltpu.SemaphoreType.DMA,
        pltpu.SemaphoreType.DMA,
    ],
)
def _emb(x_hbm, p_hbm, tok_hbm, pos_hbm, out_hbm, xi, pi, tr, pr, s1, s2):
    wid = lax.axis_index("s") * NC + lax.axis_index("c")
    base = wid * TPW
    pltpu.sync_copy(x_hbm.at[wid], xi)
    pltpu.sync_copy(p_hbm.at[wid], pi)

    def tile(g, carry):
        ct = pltpu.async_copy(tok_hbm.at[xi.at[g]], tr, s1)
        cp = pltpu.async_copy(pos_hbm.at[pi.at[g]], pr, s2)
        ct.wait()
        cp.wait()

        def addrow(r, c):
            for j in range(D // 16):
                sl = pl.ds(j * 16, 16)
                tr[r, sl] = tr[r, sl] + pr[r, sl]
            return c

        lax.fori_loop(0, G, addrow, 0)
        pltpu.sync_copy(tr, out_hbm.at[pl.ds(base + g * G, G)])
        return carry

    lax.fori_loop(0, NG, tile, 0)


def kernel(x, pos, token_table, pos_table):
    xf = x.reshape(NW, NG, G).astype(jnp.int32)
    pf = pos.reshape(NW, NG, G).astype(jnp.int32)
    out = _emb(xf, pf, token_table, pos_table)
    return out.reshape(B, L, D)
